# Initial kernel scaffold; baseline (speedup 1.0000x reference)
import jax
import jax.numpy as jnp
from jax.experimental import pallas as pl


def kernel(x, enc_W, enc_b, gate_W, expert_W, expert_b, mlp_W, mlp_b, coef_W, coef_b, dec_W, dec_b):
    raise NotImplementedError("write your pallas kernel here")


# speedup vs baseline: 1.2951x; 1.2951x over previous
import functools

import jax
import jax.numpy as jnp
from jax.experimental import pallas as pl
from jax.experimental.pallas import tpu as pltpu

TOKENS = 4096
D = 768
NE = 8
BLK = 512
NT = TOKENS // BLK


def _fused_body(x_ref, encw_ref, encb_ref, gatew_ref, expw_ref, expb_ref,
                mlpw_ref, mlpb_ref, coefw_ref, coefb_ref, decw_ref, decb_ref,
                out_ref, h_s, acc_s, scale_s, idx_s):
    e = pl.program_id(1)

    @pl.when(e == 0)
    def _init():
        h = jnp.maximum(x_ref[...] @ encw_ref[...] + encb_ref[...], 0.0)
        h_s[...] = h
        logits = h @ gatew_ref[...]
        m = jnp.max(logits, axis=-1, keepdims=True)
        ssum = jnp.sum(jnp.exp(logits - m), axis=-1, keepdims=True)
        top_gate = 1.0 / ssum
        lane = jax.lax.broadcasted_iota(jnp.int32, logits.shape, 1)
        idx = jnp.min(jnp.where(logits == m, lane, NE), axis=-1, keepdims=True)
        idx_s[...] = idx
        z = h @ coefw_ref[...] + coefb_ref[...]
        zm = jnp.max(z, axis=-1, keepdims=True)
        ze = jnp.exp(z - zm)
        c = ze / jnp.sum(ze, axis=-1, keepdims=True)
        scale_s[...] = top_gate * c[:, 0:1]
        acc_s[...] = (h @ mlpw_ref[...] + mlpb_ref[...]) * c[:, 1:2]

    ye = h_s[...] @ expw_ref[0] + expb_ref[0]
    mask = idx_s[...] == e
    acc_s[...] += jnp.where(mask, ye * scale_s[...], 0.0)

    @pl.when(e == NE - 1)
    def _fin():
        out_ref[...] = acc_s[...] @ decw_ref[...] + decb_ref[...]


def kernel(x, enc_W, enc_b, gate_W, expert_W, expert_b, mlp_W, mlp_b,
           coef_W, coef_b, dec_W, dec_b):
    grid = (NT, NE)
    full = lambda r, c: pl.BlockSpec((r, c), lambda i, e: (0, 0))
    out = pl.pallas_call(
        _fused_body,
        grid=grid,
        in_specs=[
            pl.BlockSpec((BLK, D), lambda i, e: (i, 0)),
            full(D, D),
            full(1, D),
            full(D, NE),
            pl.BlockSpec((1, D, D), lambda i, e: (e, 0, 0)),
            pl.BlockSpec((1, 1, D), lambda i, e: (e, 0, 0)),
            full(D, D),
            full(1, D),
            full(D, 2),
            full(1, 2),
            full(D, D),
            full(1, D),
        ],
        out_specs=pl.BlockSpec((BLK, D), lambda i, e: (i, 0)),
        out_shape=jax.ShapeDtypeStruct((TOKENS, D), jnp.float32),
        scratch_shapes=[
            pltpu.VMEM((BLK, D), jnp.float32),
            pltpu.VMEM((BLK, D), jnp.float32),
            pltpu.VMEM((BLK, 1), jnp.float32),
            pltpu.VMEM((BLK, 1), jnp.int32),
        ],
    )(x, enc_W, enc_b.reshape(1, D), gate_W, expert_W, expert_b.reshape(NE, 1, D),
      mlp_W, mlp_b.reshape(1, D), coef_W, coef_b.reshape(1, 2),
      dec_W, dec_b.reshape(1, D))
    return out
